# baseline (speedup 1.0000x reference)
import jax
import jax.numpy as jnp
from jax.experimental import pallas as pl

_N = 320000
_D = 128
_BLK = 3200


def _copy_block(feat_ref, out_ref):
    out_ref[...] = feat_ref[...]


def kernel(ref_bxyz, ref_feat, group_ids):
    del ref_bxyz, group_ids
    n, d = ref_feat.shape
    grid = n // _BLK
    return pl.pallas_call(
        _copy_block,
        grid=(grid,),
        in_specs=[pl.BlockSpec((_BLK, d), lambda i: (i, 0))],
        out_specs=pl.BlockSpec((_BLK, d), lambda i: (i, 0)),
        out_shape=jax.ShapeDtypeStruct((n, d), ref_feat.dtype),
    )(ref_feat)
